# baseline (speedup 1.0000x reference)
import functools

import jax
import jax.numpy as jnp
from jax import lax
from jax.experimental import pallas as pl
from jax.experimental.pallas import tpu as pltpu
from jax.experimental.pallas import tpu_sc as plsc

_NUM_FIELDS = 26
_VOCAB = 100000
_EMB_DIM = 32
_BATCH = 16384
_LANES = 16
_COL_CH = 4096
_N_CC = _BATCH // _COL_CH


def _sc_kernel(attrsT_hbm, tablesT_hbm, outT_hbm,
               row_v, col_a, col_b, acc_v, sem_a, sem_b):
    d = lax.axis_index("c") * 16 + lax.axis_index("s")
    cols = (col_a, col_b)
    sems = (sem_a, sem_b)

    @plsc.parallel_loop(0, _BATCH // _LANES, unroll=8)
    def zero_body(g):
        acc_v[pl.ds(g * _LANES, _LANES)] = jnp.zeros((_LANES,), jnp.float32)

    def field_body(i, carry):
        cps = {}
        cps[0] = pltpu.async_copy(
            attrsT_hbm.at[i, pl.ds(0, _COL_CH)], cols[0], sems[0])
        pltpu.sync_copy(tablesT_hbm.at[i, d], row_v)
        for cc in range(_N_CC):
            b0 = cc * _COL_CH
            if cc + 1 < _N_CC:
                cps[cc + 1] = pltpu.async_copy(
                    attrsT_hbm.at[i, pl.ds((cc + 1) * _COL_CH, _COL_CH)],
                    cols[(cc + 1) % 2], sems[(cc + 1) % 2])
            cps.pop(cc).wait()
            col_v = cols[cc % 2]

            @plsc.parallel_loop(0, _COL_CH // _LANES, unroll=64)
            def group_body(g, *, col_v=col_v, b0=b0):
                off = g * _LANES
                v16 = col_v[pl.ds(off, _LANES)]
                val = plsc.load_gather(row_v, [v16])
                plsc.addupdate(acc_v.at[pl.ds(b0 + off, _LANES)], val)

        return carry

    lax.fori_loop(0, _NUM_FIELDS, field_body, 0)
    pltpu.sync_copy(acc_v, outT_hbm.at[d])


@jax.jit
def kernel(attrs, tables):
    attrsT = attrs.astype(jnp.int32).T
    tablesT = jnp.transpose(tables, (0, 2, 1))
    run = functools.partial(
        pl.kernel,
        mesh=plsc.VectorSubcoreMesh(core_axis_name="c", subcore_axis_name="s"),
        compiler_params=pltpu.CompilerParams(needs_layout_passes=False),
        out_type=jax.ShapeDtypeStruct((_EMB_DIM, _BATCH), jnp.float32),
        scratch_types=[
            pltpu.VMEM((_VOCAB,), jnp.float32),
            pltpu.VMEM((_COL_CH,), jnp.int32),
            pltpu.VMEM((_COL_CH,), jnp.int32),
            pltpu.VMEM((_BATCH,), jnp.float32),
            pltpu.SemaphoreType.DMA,
            pltpu.SemaphoreType.DMA,
        ],
    )(_sc_kernel)
    outT = run(attrsT, tablesT)
    return outT.T


# speedup vs baseline: 1.0563x; 1.0563x over previous
import functools

import jax
import jax.numpy as jnp
from jax import lax
from jax.experimental import pallas as pl
from jax.experimental.pallas import tpu as pltpu
from jax.experimental.pallas import tpu_sc as plsc

_NUM_FIELDS = 26
_VOCAB = 100000
_EMB_DIM = 32
_BATCH = 16384
_LANES = 16
_COL_CH = 4096
_N_CC = _BATCH // _COL_CH


def _sc_kernel(attrsT_hbm, tablesT_hbm, outT_hbm,
               row_v, col_a, col_b, acc_v, sem_a, sem_b):
    d = lax.axis_index("c") * 16 + lax.axis_index("s")
    cols = (col_a, col_b)
    sems = (sem_a, sem_b)

    @plsc.parallel_loop(0, _BATCH // _LANES, unroll=8)
    def zero_body(g):
        acc_v[pl.ds(g * _LANES, _LANES)] = jnp.zeros((_LANES,), jnp.float32)

    def field_body(i, carry):
        cps = {}
        cps[0] = pltpu.async_copy(
            attrsT_hbm.at[i, pl.ds(0, _COL_CH)], cols[0], sems[0])
        pltpu.sync_copy(tablesT_hbm.at[i, d], row_v)
        for cc in range(_N_CC):
            b0 = cc * _COL_CH
            if cc + 1 < _N_CC:
                cps[cc + 1] = pltpu.async_copy(
                    attrsT_hbm.at[i, pl.ds((cc + 1) * _COL_CH, _COL_CH)],
                    cols[(cc + 1) % 2], sems[(cc + 1) % 2])
            cps.pop(cc).wait()
            col_v = cols[cc % 2]

            @plsc.parallel_loop(0, _COL_CH // _LANES, unroll=32)
            def group_body(g, *, col_v=col_v, b0=b0):
                off = g * _LANES
                v16 = col_v[pl.ds(off, _LANES)]
                val = plsc.load_gather(row_v, [v16])
                plsc.addupdate(acc_v.at[pl.ds(b0 + off, _LANES)], val)

        return carry

    lax.fori_loop(0, _NUM_FIELDS, field_body, 0)
    pltpu.sync_copy(acc_v, outT_hbm.at[d])


@jax.jit
def kernel(attrs, tables):
    attrsT = attrs.astype(jnp.int32).T
    tablesT = jnp.transpose(tables, (0, 2, 1))
    run = functools.partial(
        pl.kernel,
        mesh=plsc.VectorSubcoreMesh(core_axis_name="c", subcore_axis_name="s"),
        compiler_params=pltpu.CompilerParams(needs_layout_passes=False),
        out_type=jax.ShapeDtypeStruct((_EMB_DIM, _BATCH), jnp.float32),
        scratch_types=[
            pltpu.VMEM((_VOCAB,), jnp.float32),
            pltpu.VMEM((_COL_CH,), jnp.int32),
            pltpu.VMEM((_COL_CH,), jnp.int32),
            pltpu.VMEM((_BATCH,), jnp.float32),
            pltpu.SemaphoreType.DMA,
            pltpu.SemaphoreType.DMA,
        ],
    )(_sc_kernel)
    outT = run(attrsT, tablesT)
    return outT.T
